# baseline (speedup 1.0000x reference)
import functools

import jax
import jax.numpy as jnp
from jax import lax
from jax.experimental import pallas as pl
from jax.experimental.pallas import tpu as pltpu
from jax.experimental.pallas import tpu_sc as plsc

N_NODE = 10000
D = 128
E_TOT = 320000
ED = 4

NC = 1
NS = 16
NW = NC * NS
E_PW = E_TOT // NW
K = 40
SB = 2000
NSB = E_PW // SB
CPS = SB // K
NCH = E_PW // K
NTRI = (NCH - 2) // 3
NB = 3
N_PAD = 10240
ROWS_PT = N_PAD // NS
ZROWS = 64
WROWS = 128

_mesh = plsc.VectorSubcoreMesh(core_axis_name="c", subcore_axis_name="s",
                               num_cores=NC)


@functools.partial(
    pl.kernel,
    out_type=jax.ShapeDtypeStruct((NC, N_PAD, D), jnp.float32),
    mesh=_mesh,
    scratch_types=[
        pltpu.VMEM((SB,), jnp.int32),
        pltpu.VMEM((SB,), jnp.int32),
        [pltpu.VMEM((K, D), jnp.float32) for _ in range(NB)],
        [pltpu.VMEM((K, D), jnp.float32) for _ in range(NB)],
        [pltpu.VMEM((K,), jnp.int32) for _ in range(NB)],
        [pltpu.VMEM((K,), jnp.int32) for _ in range(NB)],
        pltpu.VMEM((ZROWS, D), jnp.float32),
        pltpu.VMEM_SHARED((N_PAD, D), jnp.float32),
        [pltpu.SemaphoreType.DMA for _ in range(NB)],
        [pltpu.SemaphoreType.DMA for _ in range(NB)],
        [pltpu.SemaphoreType.DMA for _ in range(NB)],
    ],
)
def _gine_scatter(x_hbm, src_hbm, dst_hbm, e_hbm, out_hbm,
                  sidx_v, didx_v, rows, xr, sk, dk, zbuf_v, accum,
                  gsem, esem, ssem):
    cid = lax.axis_index("c")
    sid = lax.axis_index("s")
    wid = cid * NS + sid
    ebase = wid * E_PW

    zero16 = jnp.zeros((16,), jnp.float32)

    def _zrow(i, _):
        for j in range(D // 16):
            zbuf_v[i, pl.ds(j * 16, 16)] = zero16
        return 0
    lax.fori_loop(0, ZROWS, _zrow, 0)
    for q in range(ROWS_PT // ZROWS):
        pltpu.sync_copy(zbuf_v, accum.at[pl.ds(sid * ROWS_PT + q * ZROWS, ZROWS)])

    plsc.subcore_barrier()

    def _stage_sb(c):
        sbase = ebase + (c // CPS) * SB
        pltpu.sync_copy(src_hbm.at[pl.ds(sbase, SB)], sidx_v)
        pltpu.sync_copy(dst_hbm.at[pl.ds(sbase, SB)], didx_v)

    def _launch(c, r):
        base = (c % CPS) * K
        for t in range((K + 15) // 16):
            o = min(t * 16, K - 16)
            sk[r][pl.ds(o, 16)] = sidx_v[pl.ds(base + o, 16)]
            dk[r][pl.ds(o, 16)] = didx_v[pl.ds(base + o, 16)]
        pltpu.async_copy(e_hbm.at[pl.ds(ebase + c * K, K)], rows[r], esem[r])
        pltpu.async_copy(x_hbm.at[sk[r]], xr[r], gsem[r])

    def _compute(r):
        def _e8(e8, _):
            for i in range(8):
                e = e8 * 8 + i
                for j in range(D // 16):
                    sl = pl.ds(j * 16, 16)
                    rows[r][e, sl] = jnp.maximum(
                        rows[r][e, sl] + xr[r][e, sl], 0.0)
            return 0
        lax.fori_loop(0, K // 8, _e8, 0)

    def _process(c, r, launch_ahead):
        pltpu.make_async_copy(e_hbm.at[pl.ds(ebase + c * K, K)], rows[r],
                              esem[r]).wait()
        pltpu.make_async_copy(x_hbm.at[sk[r]], xr[r], gsem[r]).wait()
        _compute(r)
        pltpu.async_copy(rows[r], accum.at[dk[r]], ssem[r], add=True)
        if launch_ahead:
            r2 = (r + 2) % NB
            @pl.when(c >= 1)
            def _():
                pltpu.make_async_copy(rows[r2], accum.at[dk[r2]],
                                      ssem[r2]).wait()
            c2 = c + 2
            @pl.when(c2 % CPS == 0)
            def _():
                _stage_sb(c2)
            _launch(c2, r2)

    _stage_sb(0)
    _launch(0, 0)
    _launch(1, 1)

    def _triple(t, _):
        for r in range(NB):
            _process(3 * t + r, r, True)
        return 0
    lax.fori_loop(0, NTRI, _triple, 0)
    _process(NCH - 2, (NCH - 2) % NB, False)
    _process(NCH - 1, (NCH - 1) % NB, False)
    for r in range(NB):
        pltpu.make_async_copy(rows[r], accum.at[dk[r]], ssem[r]).wait()

    plsc.subcore_barrier()

    for q in range(ROWS_PT // WROWS):
        r0 = sid * ROWS_PT + q * WROWS
        pltpu.sync_copy(accum.at[pl.ds(r0, WROWS)],
                        out_hbm.at[cid, pl.ds(r0, WROWS)])


BE = 8000
BLK = 1000


def _eproj_body(at_ref, we_ref, be_ref, o_ref):
    dn = (((1,), (0,)), ((), ()))
    o_ref[...] = lax.dot_general(at_ref[...], we_ref[...], dn,
                                 preferred_element_type=jnp.float32) + be_ref[...]


_eproj = pl.pallas_call(
    _eproj_body,
    grid=(E_TOT // BE,),
    in_specs=[
        pl.BlockSpec((BE, ED), lambda i: (i, 0)),
        pl.BlockSpec((ED, D), lambda i: (0, 0)),
        pl.BlockSpec((1, D), lambda i: (0, 0)),
    ],
    out_specs=pl.BlockSpec((BE, D), lambda i: (i, 0)),
    out_shape=jax.ShapeDtypeStruct((E_TOT, D), jnp.float32),
)


def _mlp_ln_body(x_ref, a_ref, w1_ref, b1_ref, w2_ref, b2_ref, g_ref, bt_ref,
                 o_ref):
    x = x_ref[...]
    h = x
    for c in range(NC):
        h = h + a_ref[c]
    t = jnp.dot(h, w1_ref[...], preferred_element_type=jnp.float32) + b1_ref[...]
    t = jnp.dot(jnp.maximum(t, 0.0), w2_ref[...],
                preferred_element_type=jnp.float32) + b2_ref[...]
    r = x + t
    mu = jnp.mean(r, axis=1, keepdims=True)
    var = jnp.mean((r - mu) ** 2, axis=1, keepdims=True)
    o_ref[...] = (r - mu) * lax.rsqrt(var + 1e-5) * g_ref[...] + bt_ref[...]


_mlp_ln = pl.pallas_call(
    _mlp_ln_body,
    grid=(N_NODE // BLK,),
    in_specs=[
        pl.BlockSpec((BLK, D), lambda i: (i, 0)),
        pl.BlockSpec((NC, BLK, D), lambda i: (0, i, 0)),
        pl.BlockSpec((D, D), lambda i: (0, 0)),
        pl.BlockSpec((1, D), lambda i: (0, 0)),
        pl.BlockSpec((D, D), lambda i: (0, 0)),
        pl.BlockSpec((1, D), lambda i: (0, 0)),
        pl.BlockSpec((1, D), lambda i: (0, 0)),
        pl.BlockSpec((1, D), lambda i: (0, 0)),
    ],
    out_specs=pl.BlockSpec((BLK, D), lambda i: (i, 0)),
    out_shape=jax.ShapeDtypeStruct((N_NODE, D), jnp.float32),
)


def kernel(x_var, x_constr, edge_index_v2c, edge_index_c2v, edge_attr,
           We_v, be_v, W1_v, b1_v, W2_v, b2_v,
           We_c, be_c, W1_c, b1_c, W2_c, b2_c,
           g_var, bt_var, g_constr, bt_constr):
    s_v2c = edge_index_v2c[0].astype(jnp.int32)
    d_v2c = edge_index_v2c[1].astype(jnp.int32)
    s_c2v = edge_index_c2v[0].astype(jnp.int32)
    d_c2v = edge_index_c2v[1].astype(jnp.int32)

    r1 = lambda v: v.reshape(1, D)

    attr = edge_attr.astype(jnp.float32)
    e_v = _eproj(attr, We_v, r1(be_v))

    agg_c = _gine_scatter(x_var, s_v2c, d_v2c, e_v)
    e_c = _eproj(attr, We_c, r1(be_c))
    xc = _mlp_ln(x_constr, agg_c, W1_v, r1(b1_v), W2_v, r1(b2_v),
                 r1(g_constr), r1(bt_constr))
    agg_v = _gine_scatter(xc, s_c2v, d_c2v, e_c)
    xv = _mlp_ln(x_var, agg_v, W1_c, r1(b1_c), W2_c, r1(b2_c),
                 r1(g_var), r1(bt_var))
    return (xv, xc)
